# Initial kernel scaffold; baseline (speedup 1.0000x reference)
import jax
import jax.numpy as jnp
from jax.experimental import pallas as pl


def kernel(features, W1, b1, W2, b2):
    raise NotImplementedError("write your pallas kernel here")


# speedup vs baseline: 18.0830x; 18.0830x over previous
import functools

import jax
import jax.numpy as jnp
from jax.experimental import pallas as pl

_N = 8192
_D = 256
_KEEP = 21
_BM = 256


def _emb_body(f_ref, w1_ref, b1_ref, w2_ref, b2_ref, out_ref):
    x = jax.lax.dot_general(
        f_ref[...], w1_ref[...], (((1,), (1,)), ((), ())),
        preferred_element_type=jnp.float32)
    x = jnp.maximum(x + b1_ref[...], 0.0)
    x = jax.lax.dot_general(
        x, w2_ref[...], (((1,), (1,)), ((), ())),
        preferred_element_type=jnp.float32)
    x = x + b2_ref[...]
    norm = jnp.sqrt(jnp.sum(x * x, axis=1, keepdims=True))
    out_ref[...] = x / jnp.maximum(norm, 1e-12)


def _sim_topk_body(rows_ref, emb_ref, out_ref):
    sim = jax.lax.dot_general(
        rows_ref[...], emb_ref[...], (((1,), (1,)), ((), ())),
        preferred_element_type=jnp.float32)
    cur = sim
    thresh = None
    for _ in range(_KEEP):
        thresh = jnp.max(cur, axis=1, keepdims=True)
        cur = jnp.where(cur == thresh, -jnp.inf, cur)
    out_ref[...] = jnp.where(sim >= thresh, jnp.maximum(sim, 0.0), 0.0)


@functools.partial(jax.jit, static_argnames=("interpret",))
def kernel(features, W1, b1, W2, b2, interpret=False):
    n, d = features.shape
    emb = pl.pallas_call(
        _emb_body,
        out_shape=jax.ShapeDtypeStruct((n, d), jnp.float32),
        interpret=interpret,
    )(features, W1, b1.reshape(1, d), W2, b2.reshape(1, d))

    num_blocks = n // _BM
    out = pl.pallas_call(
        _sim_topk_body,
        grid=(num_blocks,),
        in_specs=[
            pl.BlockSpec((_BM, d), lambda i: (i, 0)),
            pl.BlockSpec((n, d), lambda i: (0, 0)),
        ],
        out_specs=pl.BlockSpec((_BM, n), lambda i: (i, 0)),
        out_shape=jax.ShapeDtypeStruct((n, n), jnp.float32),
        interpret=interpret,
    )(emb, emb)
    return out


# speedup vs baseline: 41.2404x; 2.2806x over previous
import functools

import jax
import jax.numpy as jnp
from jax.experimental import pallas as pl

_N = 8192
_D = 256
_KEEP = 21
_BM = 256


def _emb_body(f_ref, w1_ref, b1_ref, w2_ref, b2_ref, out_ref):
    x = jax.lax.dot_general(
        f_ref[...], w1_ref[...], (((1,), (1,)), ((), ())),
        preferred_element_type=jnp.float32)
    x = jnp.maximum(x + b1_ref[...], 0.0)
    x = jax.lax.dot_general(
        x, w2_ref[...], (((1,), (1,)), ((), ())),
        preferred_element_type=jnp.float32)
    x = x + b2_ref[...]
    norm = jnp.sqrt(jnp.sum(x * x, axis=1, keepdims=True))
    out_ref[...] = x / jnp.maximum(norm, 1e-12)


def _full_select(sim):
    cur = sim
    thresh = None
    for _ in range(_KEEP):
        thresh = jnp.max(cur, axis=1, keepdims=True)
        cur = jnp.where(cur == thresh, -jnp.inf, cur)
    return thresh


_NC = 128
_NT = 4


def _fold_max(x):
    w = x.shape[1]
    while w > _NC:
        w //= 2
        x = jnp.maximum(x[:, :w], x[:, w:])
    return x


def _fold_sum(x):
    w = x.shape[1]
    while w > _NC:
        w //= 2
        x = x[:, :w] + x[:, w:]
    return x


def _sim_topk_body(rows_ref, emb_ref, out_ref):
    sim = jax.lax.dot_general(
        rows_ref[...], emb_ref[...], (((1,), (1,)), ((), ())),
        preferred_element_type=jnp.float32)
    bm, n = sim.shape
    reps = n // _NC
    cur = sim
    tops = []
    for t in range(_NT):
        c = _fold_max(cur)
        tops.append(c)
        if t + 1 < _NT:
            cur = jnp.where(cur == jnp.tile(c, (1, reps)), -jnp.inf, cur)
    cand = jnp.concatenate(tops, axis=1)
    cu = cand
    thresh = None
    for _ in range(_KEEP):
        thresh = jnp.max(cu, axis=1, keepdims=True)
        cu = jnp.where(cu == thresh, -jnp.inf, cu)
    cnt = _fold_sum((sim >= thresh).astype(jnp.float32))
    viol = jnp.any(cnt > float(_NT))
    thresh = jax.lax.cond(viol, _full_select, lambda s: thresh, sim)
    out_ref[...] = jnp.where(sim >= thresh, jnp.maximum(sim, 0.0), 0.0)


@functools.partial(jax.jit, static_argnames=("interpret",))
def kernel(features, W1, b1, W2, b2, interpret=False):
    n, d = features.shape
    emb = pl.pallas_call(
        _emb_body,
        out_shape=jax.ShapeDtypeStruct((n, d), jnp.float32),
        interpret=interpret,
    )(features, W1, b1.reshape(1, d), W2, b2.reshape(1, d))

    num_blocks = n // _BM
    out = pl.pallas_call(
        _sim_topk_body,
        grid=(num_blocks,),
        in_specs=[
            pl.BlockSpec((_BM, d), lambda i: (i, 0)),
            pl.BlockSpec((n, d), lambda i: (0, 0)),
        ],
        out_specs=pl.BlockSpec((_BM, n), lambda i: (i, 0)),
        out_shape=jax.ShapeDtypeStruct((n, n), jnp.float32),
        interpret=interpret,
    )(emb, emb)
    return out


# speedup vs baseline: 42.5784x; 1.0324x over previous
import functools

import jax
import jax.numpy as jnp
from jax.experimental import pallas as pl
from jax.experimental.pallas import tpu as pltpu

_N = 8192
_D = 256
_KEEP = 21
_BM = 256


def _emb_body(f_ref, w1_ref, b1_ref, w2_ref, b2_ref, out_ref):
    x = jax.lax.dot_general(
        f_ref[...], w1_ref[...], (((1,), (1,)), ((), ())),
        preferred_element_type=jnp.float32)
    x = jnp.maximum(x + b1_ref[...], 0.0)
    x = jax.lax.dot_general(
        x, w2_ref[...], (((1,), (1,)), ((), ())),
        preferred_element_type=jnp.float32)
    x = x + b2_ref[...]
    norm = jnp.sqrt(jnp.sum(x * x, axis=1, keepdims=True))
    out_ref[...] = x / jnp.maximum(norm, 1e-12)


def _full_select(sim):
    cur = sim
    thresh = None
    for _ in range(_KEEP):
        thresh = jnp.max(cur, axis=1, keepdims=True)
        cur = jnp.where(cur == thresh, -jnp.inf, cur)
    return thresh


_NC = 128
_NT = 4


def _fold_max(x):
    w = x.shape[1]
    while w > _NC:
        w //= 2
        x = jnp.maximum(x[:, :w], x[:, w:])
    return x


def _fold_sum(x):
    w = x.shape[1]
    while w > _NC:
        w //= 2
        x = x[:, :w] + x[:, w:]
    return x


def _sim_topk_body(rows_ref, emb_ref, out_ref):
    sim = jax.lax.dot_general(
        rows_ref[...], emb_ref[...], (((1,), (1,)), ((), ())),
        preferred_element_type=jnp.float32)
    bm, n = sim.shape
    reps = n // _NC
    cur = sim
    tops = []
    for t in range(_NT):
        c = _fold_max(cur)
        tops.append(c)
        if t + 1 < _NT:
            cur = jnp.where(cur == jnp.tile(c, (1, reps)), -jnp.inf, cur)
    cand = jnp.concatenate(tops, axis=1)
    cu = cand
    thresh = None
    for _ in range(_KEEP):
        thresh = jnp.max(cu, axis=1, keepdims=True)
        cu = jnp.where(cu == thresh, -jnp.inf, cu)
    cnt = _fold_sum((sim >= thresh).astype(jnp.float32))
    viol = jnp.any(cnt > float(_NT))
    thresh = jax.lax.cond(viol, _full_select, lambda s: thresh, sim)
    out_ref[...] = jnp.where(sim >= thresh, sim, 0.0)


@functools.partial(jax.jit, static_argnames=("interpret",))
def kernel(features, W1, b1, W2, b2, interpret=False):
    n, d = features.shape
    emb = pl.pallas_call(
        _emb_body,
        out_shape=jax.ShapeDtypeStruct((n, d), jnp.float32),
        interpret=interpret,
    )(features, W1, b1.reshape(1, d), W2, b2.reshape(1, d))

    num_blocks = n // _BM
    out = pl.pallas_call(
        _sim_topk_body,
        grid=(num_blocks,),
        in_specs=[
            pl.BlockSpec((_BM, d), lambda i: (i, 0)),
            pl.BlockSpec((n, d), lambda i: (0, 0)),
        ],
        out_specs=pl.BlockSpec((_BM, n), lambda i: (i, 0)),
        out_shape=jax.ShapeDtypeStruct((n, n), jnp.float32),
        compiler_params=pltpu.CompilerParams(
            dimension_semantics=("parallel",)),
        interpret=interpret,
    )(emb, emb)
    return out


# speedup vs baseline: 49.3356x; 1.1587x over previous
import functools

import jax
import jax.numpy as jnp
from jax.experimental import pallas as pl
from jax.experimental.pallas import tpu as pltpu

_N = 8192
_D = 256
_KEEP = 21
_BM = 256


def _emb_body(f_ref, w1_ref, b1_ref, w2_ref, b2_ref, out_ref):
    x = jax.lax.dot_general(
        f_ref[...], w1_ref[...], (((1,), (1,)), ((), ())),
        preferred_element_type=jnp.float32)
    x = jnp.maximum(x + b1_ref[...], 0.0)
    x = jax.lax.dot_general(
        x, w2_ref[...], (((1,), (1,)), ((), ())),
        preferred_element_type=jnp.float32)
    x = x + b2_ref[...]
    norm = jnp.sqrt(jnp.sum(x * x, axis=1, keepdims=True))
    out_ref[...] = x / jnp.maximum(norm, 1e-12)


def _full_select(sim):
    cur = sim
    thresh = None
    for _ in range(_KEEP):
        thresh = jnp.max(cur, axis=1, keepdims=True)
        cur = jnp.where(cur == thresh, -jnp.inf, cur)
    return thresh


_NC = 128
_NT = 4


def _fold_sum(x):
    w = x.shape[1]
    while w > _NC:
        w //= 2
        x = x[:, :w] + x[:, w:]
    return x


def _sim_topk_body(rows_ref, emb_ref, out_ref):
    sim = jax.lax.dot_general(
        rows_ref[...], emb_ref[...], (((1,), (1,)), ((), ())),
        preferred_element_type=jnp.float32)
    bm, n = sim.shape
    w = n // 2
    s1 = jnp.maximum(sim[:, :w], sim[:, w:])
    s2 = jnp.minimum(sim[:, :w], sim[:, w:])
    w //= 2
    a1, a2 = s1[:, :w], s2[:, :w]
    b1, b2 = s1[:, w:], s2[:, w:]
    lo_hi = jnp.minimum(a1, b1)
    hi_lo = jnp.maximum(a2, b2)
    lists = [jnp.maximum(a1, b1), jnp.maximum(lo_hi, hi_lo),
             jnp.minimum(lo_hi, hi_lo), jnp.minimum(a2, b2)]
    while w > _NC:
        w //= 2
        a = [x[:, :w] for x in lists]
        b = [x[:, w:] for x in lists]
        v1 = jnp.maximum(a[0], b[3])
        v2 = jnp.maximum(a[1], b[2])
        v3 = jnp.maximum(a[2], b[1])
        v4 = jnp.maximum(a[3], b[0])
        w1 = jnp.maximum(v1, v3)
        w3 = jnp.minimum(v1, v3)
        w2 = jnp.maximum(v2, v4)
        w4 = jnp.minimum(v2, v4)
        lists = [jnp.maximum(w1, w2), jnp.minimum(w1, w2),
                 jnp.maximum(w3, w4), jnp.minimum(w3, w4)]
    cand = jnp.concatenate(lists, axis=1)
    cu = cand
    thresh = None
    for _ in range(_KEEP):
        thresh = jnp.max(cu, axis=1, keepdims=True)
        cu = jnp.where(cu == thresh, -jnp.inf, cu)
    cnt = _fold_sum((sim >= thresh).astype(jnp.float32))
    viol = jnp.any(cnt > float(_NT))
    thresh = jax.lax.cond(viol, _full_select, lambda s: thresh, sim)
    out_ref[...] = jnp.where(sim >= thresh, sim, 0.0)


@functools.partial(jax.jit, static_argnames=("interpret",))
def kernel(features, W1, b1, W2, b2, interpret=False):
    n, d = features.shape
    emb = pl.pallas_call(
        _emb_body,
        out_shape=jax.ShapeDtypeStruct((n, d), jnp.float32),
        interpret=interpret,
    )(features, W1, b1.reshape(1, d), W2, b2.reshape(1, d))

    num_blocks = n // _BM
    out = pl.pallas_call(
        _sim_topk_body,
        grid=(num_blocks,),
        in_specs=[
            pl.BlockSpec((_BM, d), lambda i: (i, 0)),
            pl.BlockSpec((n, d), lambda i: (0, 0)),
        ],
        out_specs=pl.BlockSpec((_BM, n), lambda i: (i, 0)),
        out_shape=jax.ShapeDtypeStruct((n, n), jnp.float32),
        compiler_params=pltpu.CompilerParams(
            dimension_semantics=("parallel",)),
        interpret=interpret,
    )(emb, emb)
    return out


# speedup vs baseline: 49.8984x; 1.0114x over previous
import functools

import jax
import jax.numpy as jnp
from jax.experimental import pallas as pl
from jax.experimental.pallas import tpu as pltpu

_N = 8192
_D = 256
_KEEP = 21
_BM = 256


def _emb_body(f_ref, w1_ref, b1_ref, w2_ref, b2_ref, out_ref):
    x = jax.lax.dot_general(
        f_ref[...], w1_ref[...], (((1,), (1,)), ((), ())),
        preferred_element_type=jnp.float32)
    x = jnp.maximum(x + b1_ref[...], 0.0)
    x = jax.lax.dot_general(
        x, w2_ref[...], (((1,), (1,)), ((), ())),
        preferred_element_type=jnp.float32)
    x = x + b2_ref[...]
    norm = jnp.sqrt(jnp.sum(x * x, axis=1, keepdims=True))
    out_ref[...] = x / jnp.maximum(norm, 1e-12)


def _full_select(sim):
    cur = sim
    thresh = None
    for _ in range(_KEEP):
        thresh = jnp.max(cur, axis=1, keepdims=True)
        cur = jnp.where(cur == thresh, -jnp.inf, cur)
    return thresh


_NC = 128
_NT = 4


def _fold_sum(x):
    w = x.shape[1]
    while w > _NC:
        w //= 2
        x = x[:, :w] + x[:, w:]
    return x


def _sim_topk_body(rows_ref, emb_ref, out_ref):
    sim = jax.lax.dot_general(
        rows_ref[...], emb_ref[...], (((1,), (1,)), ((), ())),
        preferred_element_type=jnp.float32)
    bm, n = sim.shape
    w = n // 2
    s1 = jnp.maximum(sim[:, :w], sim[:, w:])
    s2 = jnp.minimum(sim[:, :w], sim[:, w:])
    w //= 2
    a1, a2 = s1[:, :w], s2[:, :w]
    b1, b2 = s1[:, w:], s2[:, w:]
    lo_hi = jnp.minimum(a1, b1)
    hi_lo = jnp.maximum(a2, b2)
    lists = [jnp.maximum(a1, b1), jnp.maximum(lo_hi, hi_lo),
             jnp.minimum(lo_hi, hi_lo), jnp.minimum(a2, b2)]
    while w > _NC:
        w //= 2
        a = [x[:, :w] for x in lists]
        b = [x[:, w:] for x in lists]
        v1 = jnp.maximum(a[0], b[3])
        v2 = jnp.maximum(a[1], b[2])
        v3 = jnp.maximum(a[2], b[1])
        v4 = jnp.maximum(a[3], b[0])
        w1 = jnp.maximum(v1, v3)
        w3 = jnp.minimum(v1, v3)
        w2 = jnp.maximum(v2, v4)
        w4 = jnp.minimum(v2, v4)
        lists = [jnp.maximum(w1, w2), jnp.minimum(w1, w2),
                 jnp.maximum(w3, w4), jnp.minimum(w3, w4)]
    l1, l2, l3, l4 = lists
    thresh = None
    for _ in range(_KEEP):
        thresh = jnp.max(l1, axis=1, keepdims=True)
        eq = l1 == thresh
        l1 = jnp.where(eq, l2, l1)
        l2 = jnp.where(eq, l3, l2)
        l3 = jnp.where(eq, l4, l3)
        l4 = jnp.where(eq, -jnp.inf, l4)
    cnt = jnp.sum(_fold_sum((sim >= thresh).astype(jnp.float32)),
                  axis=1, keepdims=True)
    viol = jnp.any(cnt != float(_KEEP))
    thresh = jax.lax.cond(viol, _full_select, lambda s: thresh, sim)
    out_ref[...] = jnp.where(sim >= thresh, sim, 0.0)


@functools.partial(jax.jit, static_argnames=("interpret",))
def kernel(features, W1, b1, W2, b2, interpret=False):
    n, d = features.shape
    emb = pl.pallas_call(
        _emb_body,
        out_shape=jax.ShapeDtypeStruct((n, d), jnp.float32),
        interpret=interpret,
    )(features, W1, b1.reshape(1, d), W2, b2.reshape(1, d))

    num_blocks = n // _BM
    out = pl.pallas_call(
        _sim_topk_body,
        grid=(num_blocks,),
        in_specs=[
            pl.BlockSpec((_BM, d), lambda i: (i, 0)),
            pl.BlockSpec((n, d), lambda i: (0, 0)),
        ],
        out_specs=pl.BlockSpec((_BM, n), lambda i: (i, 0)),
        out_shape=jax.ShapeDtypeStruct((n, n), jnp.float32),
        compiler_params=pltpu.CompilerParams(
            dimension_semantics=("parallel",)),
        interpret=interpret,
    )(emb, emb)
    return out
